# baseline (device time: 15497 ns/iter reference)
import jax
import jax.numpy as jnp
from jax import lax
from jax.experimental import pallas as pl
from jax.experimental.pallas import tpu as pltpu

N_CHUNKS = 2
CLIP = 100.0
QSCALE = 127.0 / CLIP
DEQ = CLIP / 127.0


def kernel(x, dy):
    k_per, m = x.shape
    _, n = dy.shape
    m_half = m // 2
    n_chunk = n // N_CHUNKS

    def body(x_ref, dy_ref, out_ref,
             x_bf, keep_buf, send_buf, recv_buf,
             send_sems, recv_sems):
        my_x = lax.axis_index("x")
        my_y = lax.axis_index("y")
        my_z = lax.axis_index("z")
        partner = (my_x, my_y, 1 - my_z)

        barrier_sem = pltpu.get_barrier_semaphore()
        pl.semaphore_signal(
            barrier_sem, inc=1,
            device_id=partner, device_id_type=pl.DeviceIdType.MESH,
        )
        pl.semaphore_wait(barrier_sem, 1)

        x_bf[...] = x_ref[...].astype(jnp.bfloat16)

        def quant(p):
            q = jnp.clip(p * QSCALE, -127.0, 127.0)
            return jnp.round(q).astype(jnp.int8)

        rdmas = []
        for c in range(N_CHUNKS):
            dv = dy_ref[:, c * n_chunk:(c + 1) * n_chunk].astype(jnp.bfloat16)
            p_full = lax.dot_general(
                x_bf[...], dv, (((0,), (0,)), ((), ())),
                preferred_element_type=jnp.float32,
            )

            @pl.when(my_z == 0)
            def _(c=c, p_full=p_full):
                send_buf[:, c * n_chunk:(c + 1) * n_chunk] = quant(
                    p_full[m_half:, :])
                keep_buf[:, c * n_chunk:(c + 1) * n_chunk] = p_full[:m_half, :]

            @pl.when(my_z == 1)
            def _(c=c, p_full=p_full):
                send_buf[:, c * n_chunk:(c + 1) * n_chunk] = quant(
                    p_full[:m_half, :])
                keep_buf[:, c * n_chunk:(c + 1) * n_chunk] = p_full[m_half:, :]

            rdma = pltpu.make_async_remote_copy(
                src_ref=send_buf.at[:, pl.ds(c * n_chunk, n_chunk)],
                dst_ref=recv_buf.at[:, pl.ds(c * n_chunk, n_chunk)],
                send_sem=send_sems.at[c],
                recv_sem=recv_sems.at[c],
                device_id=partner,
                device_id_type=pl.DeviceIdType.MESH,
            )
            rdma.start()
            rdmas.append(rdma)

            if c >= 2:
                cc = c - 2
                rdmas[cc].wait_recv()
                out_ref[:, cc * n_chunk:(cc + 1) * n_chunk] = (
                    keep_buf[:, cc * n_chunk:(cc + 1) * n_chunk]
                    + recv_buf[:, cc * n_chunk:(cc + 1) * n_chunk].astype(
                        jnp.float32) * DEQ)

        for cc in range(N_CHUNKS - 2, N_CHUNKS):
            rdmas[cc].wait_recv()
            out_ref[:, cc * n_chunk:(cc + 1) * n_chunk] = (
                keep_buf[:, cc * n_chunk:(cc + 1) * n_chunk]
                + recv_buf[:, cc * n_chunk:(cc + 1) * n_chunk].astype(
                    jnp.float32) * DEQ)


        for c in range(N_CHUNKS):
            rdmas[c].wait_send()

    return pl.pallas_call(
        body,
        out_shape=jax.ShapeDtypeStruct((m_half, n), jnp.float32),
        in_specs=[
            pl.BlockSpec(memory_space=pltpu.VMEM),
            pl.BlockSpec(memory_space=pltpu.VMEM),
        ],
        out_specs=pl.BlockSpec(memory_space=pltpu.VMEM),
        scratch_shapes=[
            pltpu.VMEM((k_per, m), jnp.bfloat16),
            pltpu.VMEM((m_half, n), jnp.float32),
            pltpu.VMEM((m_half, n), jnp.int8),
            pltpu.VMEM((m_half, n), jnp.int8),
            pltpu.SemaphoreType.DMA((N_CHUNKS,)),
            pltpu.SemaphoreType.DMA((N_CHUNKS,)),
        ],
        compiler_params=pltpu.CompilerParams(collective_id=0),
    )(x, dy)


# device time: 15083 ns/iter; 1.0274x vs baseline; 1.0274x over previous
import jax
import jax.numpy as jnp
from jax import lax
from jax.experimental import pallas as pl
from jax.experimental.pallas import tpu as pltpu

N_CHUNKS = 4
CLIP = 100.0
QSCALE = 127.0 / CLIP
DEQ = CLIP / 127.0


def kernel(x, dy):
    k_per, m = x.shape
    _, n = dy.shape
    m_half = m // 2
    n_chunk = n // N_CHUNKS

    def body(x_ref, dy_ref, out_ref,
             x_bf, keep_buf, send_buf, recv_buf,
             send_sems, recv_sems):
        my_x = lax.axis_index("x")
        my_y = lax.axis_index("y")
        my_z = lax.axis_index("z")
        partner = (my_x, my_y, 1 - my_z)

        barrier_sem = pltpu.get_barrier_semaphore()
        pl.semaphore_signal(
            barrier_sem, inc=1,
            device_id=partner, device_id_type=pl.DeviceIdType.MESH,
        )
        pl.semaphore_wait(barrier_sem, 1)

        x_bf[...] = x_ref[...].astype(jnp.bfloat16)

        def quant(p):
            q = jnp.clip(p * QSCALE, -127.0, 127.0)
            return jnp.round(q).astype(jnp.int8)

        rdmas = []
        for c in range(N_CHUNKS):
            dv = dy_ref[:, c * n_chunk:(c + 1) * n_chunk].astype(jnp.bfloat16)
            p_full = lax.dot_general(
                x_bf[...], dv, (((0,), (0,)), ((), ())),
                preferred_element_type=jnp.float32,
            )

            @pl.when(my_z == 0)
            def _(c=c, p_full=p_full):
                send_buf[:, c * n_chunk:(c + 1) * n_chunk] = quant(
                    p_full[m_half:, :])
                keep_buf[:, c * n_chunk:(c + 1) * n_chunk] = p_full[:m_half, :]

            @pl.when(my_z == 1)
            def _(c=c, p_full=p_full):
                send_buf[:, c * n_chunk:(c + 1) * n_chunk] = quant(
                    p_full[:m_half, :])
                keep_buf[:, c * n_chunk:(c + 1) * n_chunk] = p_full[m_half:, :]

            rdma = pltpu.make_async_remote_copy(
                src_ref=send_buf.at[:, pl.ds(c * n_chunk, n_chunk)],
                dst_ref=recv_buf.at[:, pl.ds(c * n_chunk, n_chunk)],
                send_sem=send_sems.at[c],
                recv_sem=recv_sems.at[c],
                device_id=partner,
                device_id_type=pl.DeviceIdType.MESH,
            )
            rdma.start()
            rdmas.append(rdma)

            if c >= 2:
                cc = c - 2
                rdmas[cc].wait_recv()
                out_ref[:, cc * n_chunk:(cc + 1) * n_chunk] = (
                    keep_buf[:, cc * n_chunk:(cc + 1) * n_chunk]
                    + recv_buf[:, cc * n_chunk:(cc + 1) * n_chunk].astype(
                        jnp.float32) * DEQ)

        for cc in range(N_CHUNKS - 2, N_CHUNKS):
            rdmas[cc].wait_recv()
            out_ref[:, cc * n_chunk:(cc + 1) * n_chunk] = (
                keep_buf[:, cc * n_chunk:(cc + 1) * n_chunk]
                + recv_buf[:, cc * n_chunk:(cc + 1) * n_chunk].astype(
                    jnp.float32) * DEQ)


        for c in range(N_CHUNKS):
            rdmas[c].wait_send()

    return pl.pallas_call(
        body,
        out_shape=jax.ShapeDtypeStruct((m_half, n), jnp.float32),
        in_specs=[
            pl.BlockSpec(memory_space=pltpu.VMEM),
            pl.BlockSpec(memory_space=pltpu.VMEM),
        ],
        out_specs=pl.BlockSpec(memory_space=pltpu.VMEM),
        scratch_shapes=[
            pltpu.VMEM((k_per, m), jnp.bfloat16),
            pltpu.VMEM((m_half, n), jnp.float32),
            pltpu.VMEM((m_half, n), jnp.int8),
            pltpu.VMEM((m_half, n), jnp.int8),
            pltpu.SemaphoreType.DMA((N_CHUNKS,)),
            pltpu.SemaphoreType.DMA((N_CHUNKS,)),
        ],
        compiler_params=pltpu.CompilerParams(collective_id=0),
    )(x, dy)
